# baseline (speedup 1.0000x reference)
import math

import jax
import jax.numpy as jnp
from jax import lax
from jax.experimental import pallas as pl

_N_BLOCK = 1000
_E_BLOCK = 8000
_TIME_HALF = 16
_LOG_MAX_POS = math.log(10000.0)


def _node_kernel(h_ref, t_ref, f_ref, atom_ref, w_emb_t_ref, b_emb_ref,
                 w_in_rep_ref, b_in_rep_ref, out_ref):
    bn = h_ref.shape[1]
    hv = h_ref[0]
    tv = t_ref[0].astype(jnp.float32)

    atom_iota = lax.broadcasted_iota(jnp.int32, (bn, 100), 1)
    onehot = (hv == atom_iota).astype(jnp.float32)
    atom_embed = jnp.dot(onehot, atom_ref[...],
                         preferred_element_type=jnp.float32)

    hcat = jnp.concatenate([atom_embed, f_ref[...]], axis=1)
    h_feat = jnp.dot(hcat, w_emb_t_ref[...],
                     preferred_element_type=jnp.float32) + b_emb_ref[...]

    j = lax.broadcasted_iota(jnp.int32, (bn, 2 * _TIME_HALF), 1).astype(jnp.float32)
    is_cos = j >= _TIME_HALF
    jm = jnp.where(is_cos, j - _TIME_HALF, j)
    freq = jnp.exp(jm * (-_LOG_MAX_POS / (_TIME_HALF - 1)))
    arg = tv * freq + jnp.where(is_cos, jnp.float32(math.pi / 2), jnp.float32(0.0))
    t_emb = jnp.sin(arg)

    feat = jnp.concatenate([h_feat, t_emb], axis=1)
    out_ref[...] = jnp.dot(feat, w_in_rep_ref[...],
                           preferred_element_type=jnp.float32) + b_in_rep_ref[...]


def _edge_kernel(attr_ref, table_rep_ref, out_ref):
    be = attr_ref.shape[1]
    av = attr_ref[0]
    iota = lax.broadcasted_iota(jnp.int32, (be, 50), 1)
    onehot = (av == iota).astype(jnp.float32)
    out_ref[...] = jnp.dot(onehot, table_rep_ref[...],
                           preferred_element_type=jnp.float32)


def kernel(diffusion_t, x, h, f, edge_index, edge_attr, batch, atom_table,
           W_emb, b_emb, edge_table, cond_table, W_in, b_in):
    N, FT = f.shape
    E = edge_attr.shape[0]
    T = x.shape[-1]
    NODE = atom_table.shape[1]
    HID = W_in.shape[0]
    ED = edge_table.shape[1]

    nb = N // _N_BLOCK
    nbe = E // _E_BLOCK

    w_emb_t = W_emb.T
    w_in_rep = jnp.tile(W_in.T, (1, T))
    b_in_rep = jnp.tile(b_in, T)[None, :]
    table_rep = jnp.tile(edge_table, (1, T))
    b_emb_row = b_emb[None, :]

    h3 = h.astype(jnp.int32).reshape(nb, _N_BLOCK, 1)
    t3 = diffusion_t.astype(jnp.int32).reshape(nb, _N_BLOCK, 1)
    a3 = edge_attr.astype(jnp.int32).reshape(nbe, _E_BLOCK, 1)

    h2d = pl.pallas_call(
        _node_kernel,
        grid=(nb,),
        in_specs=[
            pl.BlockSpec((1, _N_BLOCK, 1), lambda i: (i, 0, 0)),
            pl.BlockSpec((1, _N_BLOCK, 1), lambda i: (i, 0, 0)),
            pl.BlockSpec((_N_BLOCK, FT), lambda i: (i, 0)),
            pl.BlockSpec(atom_table.shape, lambda i: (0, 0)),
            pl.BlockSpec(w_emb_t.shape, lambda i: (0, 0)),
            pl.BlockSpec(b_emb_row.shape, lambda i: (0, 0)),
            pl.BlockSpec(w_in_rep.shape, lambda i: (0, 0)),
            pl.BlockSpec(b_in_rep.shape, lambda i: (0, 0)),
        ],
        out_specs=pl.BlockSpec((_N_BLOCK, HID * T), lambda i: (i, 0)),
        out_shape=jax.ShapeDtypeStruct((N, HID * T), jnp.float32),
    )(h3, t3, f, atom_table, w_emb_t, b_emb_row, w_in_rep, b_in_rep)

    e2d = pl.pallas_call(
        _edge_kernel,
        grid=(nbe,),
        in_specs=[
            pl.BlockSpec((1, _E_BLOCK, 1), lambda i: (i, 0, 0)),
            pl.BlockSpec(table_rep.shape, lambda i: (0, 0)),
        ],
        out_specs=pl.BlockSpec((_E_BLOCK, ED * T), lambda i: (i, 0)),
        out_shape=jax.ShapeDtypeStruct((E, ED * T), jnp.float32),
    )(a3, table_rep)

    h_out = jnp.transpose(h2d.reshape(N, T, HID), (0, 2, 1))
    edge_out = jnp.transpose(e2d.reshape(E, T, ED), (0, 2, 1))
    return (x, h_out, edge_out)


# speedup vs baseline: 3.0040x; 3.0040x over previous
import math

import jax
import jax.numpy as jnp
from jax import lax
from jax.experimental import pallas as pl

_N_BLOCK = 1000
_E_BLOCK = 16000
_TIME_HALF = 16
_LOG_MAX_POS = math.log(10000.0)


def _node_kernel(h_ref, t_ref, f_ref, atom_ref, w_emb_t_ref, b_emb_ref,
                 w_in_rep_ref, b_in_rep_ref, out_ref):
    bn = h_ref.shape[1]
    hv = h_ref[0]
    tv = t_ref[0].astype(jnp.float32)

    atom_iota = lax.broadcasted_iota(jnp.int32, (bn, 100), 1)
    onehot = (hv == atom_iota).astype(jnp.float32)
    atom_embed = jnp.dot(onehot, atom_ref[...],
                         preferred_element_type=jnp.float32)

    hcat = jnp.concatenate([atom_embed, f_ref[...]], axis=1)
    h_feat = jnp.dot(hcat, w_emb_t_ref[...],
                     preferred_element_type=jnp.float32) + b_emb_ref[...]

    j = lax.broadcasted_iota(jnp.int32, (bn, 2 * _TIME_HALF), 1).astype(jnp.float32)
    is_cos = j >= _TIME_HALF
    jm = jnp.where(is_cos, j - _TIME_HALF, j)
    freq = jnp.exp(jm * (-_LOG_MAX_POS / (_TIME_HALF - 1)))
    arg = tv * freq + jnp.where(is_cos, jnp.float32(math.pi / 2), jnp.float32(0.0))
    t_emb = jnp.sin(arg)

    feat = jnp.concatenate([h_feat, t_emb], axis=1)
    out_ref[...] = jnp.dot(feat, w_in_rep_ref[...],
                           preferred_element_type=jnp.float32) + b_in_rep_ref[...]


def _edge_kernel(attr_ref, table_rep_t_ref, out_ref):
    be = attr_ref.shape[2]
    av = attr_ref[0]
    iota = lax.broadcasted_iota(jnp.int32, (50, be), 0)
    onehot = (iota == av).astype(jnp.float32)
    out_ref[...] = jnp.dot(table_rep_t_ref[...], onehot,
                           preferred_element_type=jnp.float32)


def kernel(diffusion_t, x, h, f, edge_index, edge_attr, batch, atom_table,
           W_emb, b_emb, edge_table, cond_table, W_in, b_in):
    N, FT = f.shape
    E = edge_attr.shape[0]
    T = x.shape[-1]
    HID = W_in.shape[0]
    ED = edge_table.shape[1]

    nb = N // _N_BLOCK
    nbe = E // _E_BLOCK

    w_emb_t = W_emb.T
    w_in_rep = jnp.tile(W_in.T, (1, T))
    b_in_rep = jnp.tile(b_in, T)[None, :]
    table_rep_t = jnp.repeat(edge_table.T, T, axis=0)
    b_emb_row = b_emb[None, :]

    h3 = h.astype(jnp.int32).reshape(nb, _N_BLOCK, 1)
    t3 = diffusion_t.astype(jnp.int32).reshape(nb, _N_BLOCK, 1)
    a3 = edge_attr.astype(jnp.int32).reshape(nbe, 1, _E_BLOCK)

    h2d = pl.pallas_call(
        _node_kernel,
        grid=(nb,),
        in_specs=[
            pl.BlockSpec((1, _N_BLOCK, 1), lambda i: (i, 0, 0)),
            pl.BlockSpec((1, _N_BLOCK, 1), lambda i: (i, 0, 0)),
            pl.BlockSpec((_N_BLOCK, FT), lambda i: (i, 0)),
            pl.BlockSpec(atom_table.shape, lambda i: (0, 0)),
            pl.BlockSpec(w_emb_t.shape, lambda i: (0, 0)),
            pl.BlockSpec(b_emb_row.shape, lambda i: (0, 0)),
            pl.BlockSpec(w_in_rep.shape, lambda i: (0, 0)),
            pl.BlockSpec(b_in_rep.shape, lambda i: (0, 0)),
        ],
        out_specs=pl.BlockSpec((_N_BLOCK, HID * T), lambda i: (i, 0)),
        out_shape=jax.ShapeDtypeStruct((N, HID * T), jnp.float32),
    )(h3, t3, f, atom_table, w_emb_t, b_emb_row, w_in_rep, b_in_rep)

    e2dt = pl.pallas_call(
        _edge_kernel,
        grid=(nbe,),
        in_specs=[
            pl.BlockSpec((1, 1, _E_BLOCK), lambda i: (i, 0, 0)),
            pl.BlockSpec(table_rep_t.shape, lambda i: (0, 0)),
        ],
        out_specs=pl.BlockSpec((ED * T, _E_BLOCK), lambda i: (0, i)),
        out_shape=jax.ShapeDtypeStruct((ED * T, E), jnp.float32),
    )(a3, table_rep_t)

    h_out = jnp.transpose(h2d.reshape(N, T, HID), (0, 2, 1))
    edge_out = jnp.transpose(e2dt.reshape(ED, T, E), (2, 0, 1))
    return (x, h_out, edge_out)
